# Initial kernel scaffold; baseline (speedup 1.0000x reference)
import jax
import jax.numpy as jnp
from jax.experimental import pallas as pl


def kernel(x_cell, x_gene, edge_index_cell, edge_index_gene, cell_W1, cell_b1, cell_W2, cell_b2, gene_W1, gene_b1, gene_W2, gene_b2, agg_W, agg_b):
    raise NotImplementedError("write your pallas kernel here")


# speedup vs baseline: 1.0495x; 1.0495x over previous
import functools

import jax
import jax.numpy as jnp
from jax import lax
from jax.experimental import pallas as pl
from jax.experimental.pallas import tpu as pltpu
from jax.experimental.pallas import tpu_sc as plsc

_SUB = 5
_H = 256
_CT = 320
_SEG = 2048
_FLUSH = 2048
_CAP = 4096 + 16


def _mesh():
    return plsc.VectorSubcoreMesh(core_axis_name="c", subcore_axis_name="s")


def _ceil_to(x, m):
    return -(-x // m) * m


def _vgather(v, idx):
    dnums = lax.GatherDimensionNumbers(offset_dims=(), collapsed_slice_dims=(0,),
                                       start_index_map=(0,))
    return lax.gather(v, idx[:, None], dnums, slice_sizes=(1,),
                      mode=lax.GatherScatterMode.PROMISE_IN_BOUNDS)


def _prefix16(m, iota):
    cs = jnp.where(m, 1, 0)
    for sh in (1, 2, 4, 8):
        shifted = _vgather(cs, jnp.maximum(iota - sh, 0))
        cs = cs + jnp.where(iota >= sh, shifted, 0)
    return cs


def _scan_loop(nsegs, src2d, dst2d, st_s, st_d, csrc, cdst, sem_s,
               mylo, myhi, iota, flush):
    def stage(g, gb):
        pltpu.async_copy(src2d.at[pl.ds(g * (_SEG // 128), _SEG // 128)],
                         st_s.at[gb], sem_s)
        pltpu.async_copy(dst2d.at[pl.ds(g * (_SEG // 128), _SEG // 128)],
                         st_d.at[gb], sem_s)

    def stage_wait(g, gb):
        pltpu.make_async_copy(src2d.at[pl.ds(g * (_SEG // 128), _SEG // 128)],
                              st_s.at[gb], sem_s).wait()
        pltpu.make_async_copy(dst2d.at[pl.ds(g * (_SEG // 128), _SEG // 128)],
                              st_d.at[gb], sem_s).wait()

    stage(0, 0)

    def seg_body(g, cursor):
        gb = lax.rem(g, 2)
        stage_wait(g, gb)

        @pl.when(g + 1 < nsegs)
        def _():
            stage(g + 1, lax.rem(g + 1, 2))

        def vreg_body(v, cur):
            vr = v >> 3
            vc = (v & 7) * 16
            d16 = st_d[gb, vr, pl.ds(vc, 16)]
            m = (d16 >= mylo) & (d16 < myhi)
            cnt = plsc.all_reduce_population_count(m)

            @pl.when(cnt[0] > 0)
            def _():
                s16 = st_s[gb, vr, pl.ds(vc, 16)]
                idx = cur + _prefix16(m, iota) - 1
                plsc.store_scatter(csrc, [idx], s16, mask=m)
                plsc.store_scatter(cdst, [idx], d16 - mylo, mask=m)

            return cur + cnt

        cursor = lax.fori_loop(0, _SEG // 16, vreg_body, cursor)

        do_flush = (cursor[0] >= _FLUSH) | (g == nsegs - 1)

        @pl.when(do_flush & (cursor[0] > 0))
        def _():
            pend = (cursor + 15) & jnp.int32(-16)
            pidx = cursor + iota
            pm = pidx < pend
            plsc.store_scatter(csrc, [pidx], jnp.zeros((16,), jnp.int32),
                               mask=pm)
            plsc.store_scatter(cdst, [pidx], jnp.full((16,), _CT, jnp.int32),
                               mask=pm)
            flush(pend[0] >> 4)

        return jnp.where(do_flush, jnp.zeros((16,), jnp.int32), cursor)

    lax.fori_loop(0, nsegs, seg_body, jnp.zeros((16,), jnp.int32))


def _make_sc_degree(E_pad, n_out, npass):
    nsegs = E_pad // _SEG

    @functools.partial(
        pl.kernel,
        out_type=jax.ShapeDtypeStruct((n_out, 16), jnp.float32),
        mesh=_mesh(),
        scratch_types=[
            pltpu.VMEM((2, _SEG // 128, 128), jnp.int32),
            pltpu.VMEM((2, _SEG // 128, 128), jnp.int32),
            pltpu.VMEM((_CAP,), jnp.int32),
            pltpu.VMEM((_CAP,), jnp.int32),
            pltpu.VMEM((_CT + 8, 16), jnp.float32),
            pltpu.SemaphoreType.DMA,
        ],
        compiler_params=pltpu.CompilerParams(needs_layout_passes=False),
    )
    def deg_kernel(src2d, dst2d, out, st_s, st_d, csrc, cdst, deg, sem_s):
        c = lax.axis_index("c")
        s = lax.axis_index("s")
        w = s * 2 + c
        iota = lax.iota(jnp.int32, 16)
        ones = jnp.ones((16,), jnp.float32)

        for p in range(npass):
            mylo = (p * 32 + w) * _CT
            myhi = mylo + _CT

            def zbody(t, carry):
                deg[t, :] = jnp.zeros((16,), jnp.float32)
                return carry

            lax.fori_loop(0, _CT + 8, zbody, 0)

            def flush(nb):
                def bat(b, carry):
                    cd16 = cdst[pl.ds(b * 16, 16)]
                    for q in range(16):
                        r = cd16[q]
                        deg[r, :] = deg[r, :] + ones
                    return carry

                lax.fori_loop(0, nb, bat, 0)

            _scan_loop(nsegs, src2d, dst2d, st_s, st_d, csrc, cdst, sem_s,
                       mylo, myhi, iota, flush)

            pltpu.sync_copy(deg.at[pl.ds(0, _CT)],
                            out.at[pl.ds(mylo, _CT)])

    return deg_kernel


def _make_sc_segsum(E_pad, n_out, npass):
    nsegs = E_pad // _SEG

    @functools.partial(
        pl.kernel,
        out_type=jax.ShapeDtypeStruct((n_out, _H), jnp.float32),
        mesh=_mesh(),
        scratch_types=[
            pltpu.VMEM((2, _SEG // 128, 128), jnp.int32),
            pltpu.VMEM((2, _SEG // 128, 128), jnp.int32),
            pltpu.VMEM((_CAP,), jnp.int32),
            pltpu.VMEM((_CAP,), jnp.int32),
            pltpu.VMEM((_CT + 8, _H), jnp.float32),
            pltpu.VMEM((2, 16, _H), jnp.float32),
            pltpu.SemaphoreType.DMA,
            pltpu.SemaphoreType.DMA,
        ],
        compiler_params=pltpu.CompilerParams(needs_layout_passes=False),
    )
    def seg_kernel(hp, src2d, dst2d, out,
                   st_s, st_d, csrc, cdst, acc, rows, sem_s, sem_g):
        c = lax.axis_index("c")
        s = lax.axis_index("s")
        w = s * 2 + c
        iota = lax.iota(jnp.int32, 16)

        for p in range(npass):
            mylo = (p * 32 + w) * _CT
            myhi = mylo + _CT

            def zbody(t, carry):
                r = t >> 4
                k = t & 15
                acc[r, pl.ds(k * 16, 16)] = jnp.zeros((16,), jnp.float32)
                return carry

            lax.fori_loop(0, (_CT + 8) * 16, zbody, 0)

            def gather(b, jb):
                cs16 = csrc[pl.ds(b * 16, 16)]
                pltpu.async_copy(hp.at[cs16], rows.at[jb], sem_g)

            def gather_wait(b, jb):
                cs16 = csrc[pl.ds(b * 16, 16)]
                pltpu.make_async_copy(hp.at[cs16], rows.at[jb],
                                      sem_g).wait()

            def flush(nb):
                gather(0, 0)

                def bat(b, carry):
                    jb = lax.rem(b, 2)
                    gather_wait(b, jb)

                    @pl.when(b + 1 < nb)
                    def _():
                        gather(b + 1, 1 - jb)

                    cd16 = cdst[pl.ds(b * 16, 16)]
                    for q in range(16):
                        r = cd16[q]
                        for f in range(_H // 16):
                            sl = pl.ds(f * 16, 16)
                            acc[r, sl] = acc[r, sl] + rows[jb, q, sl]
                    return carry

                lax.fori_loop(0, nb, bat, 0)

            _scan_loop(nsegs, src2d, dst2d, st_s, st_d, csrc, cdst, sem_s,
                       mylo, myhi, iota, flush)

            pltpu.sync_copy(acc.at[pl.ds(0, _CT)],
                            out.at[pl.ds(mylo, _CT)])

    return seg_kernel


def _dinv_block(dp_ref):
    return lax.rsqrt(1.0 + dp_ref[:, 0:1])


def _tc_prep(x, W, degp):
    n_pad, D = x.shape

    def body(x_ref, w_ref, dp_ref, hp_ref):
        dinv = _dinv_block(dp_ref)
        h = jnp.dot(x_ref[...], w_ref[...], preferred_element_type=jnp.float32)
        hp_ref[...] = h * dinv

    return pl.pallas_call(
        body,
        grid=(n_pad // 256,),
        in_specs=[
            pl.BlockSpec((256, D), lambda i: (i, 0)),
            pl.BlockSpec((D, _H), lambda i: (0, 0)),
            pl.BlockSpec((256, 16), lambda i: (i, 0)),
        ],
        out_specs=pl.BlockSpec((256, _H), lambda i: (i, 0)),
        out_shape=jax.ShapeDtypeStruct((n_pad, _H), jnp.float32),
    )(x, W, degp)


def _tc_mid(acc, hp, degp, W2, b1):
    n_pad = hp.shape[0]

    def body(a_ref, h_ref, dp_ref, w_ref, b_ref, o_ref):
        dinv = _dinv_block(dp_ref)
        z = dinv * (a_ref[...] + h_ref[...]) + b_ref[...]
        a = jnp.maximum(z, 0.0)
        o_ref[...] = dinv * jnp.dot(a, w_ref[...],
                                    preferred_element_type=jnp.float32)

    return pl.pallas_call(
        body,
        grid=(n_pad // 256,),
        in_specs=[
            pl.BlockSpec((256, _H), lambda i: (i, 0)),
            pl.BlockSpec((256, _H), lambda i: (i, 0)),
            pl.BlockSpec((256, 16), lambda i: (i, 0)),
            pl.BlockSpec((_H, _H), lambda i: (0, 0)),
            pl.BlockSpec((1, _H), lambda i: (0, 0)),
        ],
        out_specs=pl.BlockSpec((256, _H), lambda i: (i, 0)),
        out_shape=jax.ShapeDtypeStruct((n_pad, _H), jnp.float32),
    )(acc, hp, degp, W2, b1)


def _tc_final(acc, hp, degp, b2):
    n_pad = hp.shape[0]

    def body(a_ref, h_ref, dp_ref, b_ref, o_ref):
        dinv = _dinv_block(dp_ref)
        o_ref[...] = dinv * (a_ref[...] + h_ref[...]) + b_ref[...]

    return pl.pallas_call(
        body,
        grid=(n_pad // 256,),
        in_specs=[
            pl.BlockSpec((256, _H), lambda i: (i, 0)),
            pl.BlockSpec((256, _H), lambda i: (i, 0)),
            pl.BlockSpec((256, 16), lambda i: (i, 0)),
            pl.BlockSpec((1, _H), lambda i: (0, 0)),
        ],
        out_specs=pl.BlockSpec((256, _H), lambda i: (i, 0)),
        out_shape=jax.ShapeDtypeStruct((n_pad, _H), jnp.float32),
    )(acc, hp, degp, b2)


def _tc_agg(x, W, b):
    n_pad, K = x.shape

    def body(x_ref, w_ref, b_ref, o_ref):
        o_ref[...] = jnp.dot(x_ref[...], w_ref[...],
                             preferred_element_type=jnp.float32) + b_ref[...]

    return pl.pallas_call(
        body,
        grid=(n_pad // 256,),
        in_specs=[
            pl.BlockSpec((256, K), lambda i: (i, 0)),
            pl.BlockSpec((K, _H), lambda i: (0, 0)),
            pl.BlockSpec((1, _H), lambda i: (0, 0)),
        ],
        out_specs=pl.BlockSpec((256, _H), lambda i: (i, 0)),
        out_shape=jax.ShapeDtypeStruct((n_pad, _H), jnp.float32),
    )(x, W, b)


def _encode(x, edge_index, W1, b1, W2, b2):
    N = x.shape[0]
    E = edge_index.shape[1]
    n_pad = _ceil_to(N, 256)
    npass = -(-n_pad // (32 * _CT))
    n_out = npass * 32 * _CT
    E_pad = _ceil_to(E, _SEG)

    src = edge_index[0].astype(jnp.int32)
    dst = edge_index[1].astype(jnp.int32)
    src2d = jnp.pad(src, (0, E_pad - E)).reshape(E_pad // 128, 128)
    dst2d = jnp.pad(dst, (0, E_pad - E),
                    constant_values=n_out).reshape(E_pad // 128, 128)

    xp = jnp.pad(x, ((0, n_pad - N), (0, 0)))
    b1_2d = b1.reshape(1, _H)
    b2_2d = b2.reshape(1, _H)

    deg = _make_sc_degree(E_pad, n_out, npass)(src2d, dst2d)
    seg = _make_sc_segsum(E_pad, n_out, npass)

    hp1 = _tc_prep(xp, W1, deg)
    acc1 = seg(hp1, src2d, dst2d)
    hp2 = _tc_mid(acc1, hp1, deg, W2, b1_2d)
    acc2 = seg(hp2, src2d, dst2d)
    out = _tc_final(acc2, hp2, deg, b2_2d)
    return out[:N]


def kernel(x_cell, x_gene, edge_index_cell, edge_index_gene,
           cell_W1, cell_b1, cell_W2, cell_b2,
           gene_W1, gene_b1, gene_W2, gene_b2,
           agg_W, agg_b):
    gene_embeddings = _encode(x_gene, edge_index_gene,
                              gene_W1, gene_b1, gene_W2, gene_b2)
    z_cell = _encode(x_cell, edge_index_cell,
                     cell_W1, cell_b1, cell_W2, cell_b2)

    N_cell = x_cell.shape[0]
    ge_r = gene_embeddings.reshape(N_cell, _SUB * _H)
    n_pad = _ceil_to(N_cell, 256)
    ge_rp = jnp.pad(ge_r, ((0, n_pad - N_cell), (0, 0)))
    z_gene = _tc_agg(ge_rp, agg_W, agg_b.reshape(1, _H))[:N_cell]

    z_combined = jnp.concatenate((z_cell, z_gene), axis=1)
    return (z_combined, z_cell, z_gene, gene_embeddings)


# speedup vs baseline: 1.8511x; 1.7638x over previous
import functools

import jax
import jax.numpy as jnp
from jax import lax
from jax.experimental import pallas as pl
from jax.experimental.pallas import tpu as pltpu
from jax.experimental.pallas import tpu_sc as plsc

_SUB = 5
_H = 256
_CT = 400
_SEG = 1024
_FLUSH = 2048
_CAP = 3072 + 32


def _mesh():
    return plsc.VectorSubcoreMesh(core_axis_name="c", subcore_axis_name="s")


def _ceil_to(x, m):
    return -(-x // m) * m


def _vgather(v, idx):
    dnums = lax.GatherDimensionNumbers(offset_dims=(), collapsed_slice_dims=(0,),
                                       start_index_map=(0,))
    return lax.gather(v, idx[:, None], dnums, slice_sizes=(1,),
                      mode=lax.GatherScatterMode.PROMISE_IN_BOUNDS)


def _prefix16(m, iota):
    cs = jnp.where(m, 1, 0)
    for sh in (1, 2, 4, 8):
        shifted = _vgather(cs, jnp.maximum(iota - sh, 0))
        cs = cs + jnp.where(iota >= sh, shifted, 0)
    return cs


def _scan_loop(nsegs, src2d, dst2d, st_s, st_d, csrc, cdst, sem_s,
               mylo, myhi, iota, flush):
    def stage(g, gb):
        pltpu.async_copy(src2d.at[pl.ds(g * (_SEG // 128), _SEG // 128)],
                         st_s.at[gb], sem_s)
        pltpu.async_copy(dst2d.at[pl.ds(g * (_SEG // 128), _SEG // 128)],
                         st_d.at[gb], sem_s)

    def stage_wait(g, gb):
        pltpu.make_async_copy(src2d.at[pl.ds(g * (_SEG // 128), _SEG // 128)],
                              st_s.at[gb], sem_s).wait()
        pltpu.make_async_copy(dst2d.at[pl.ds(g * (_SEG // 128), _SEG // 128)],
                              st_d.at[gb], sem_s).wait()

    stage(0, 0)

    def seg_body(g, cursor):
        gb = lax.rem(g, 2)
        stage_wait(g, gb)

        @pl.when(g + 1 < nsegs)
        def _():
            stage(g + 1, lax.rem(g + 1, 2))

        def vreg_body(v, cur):
            vr = v >> 3
            vc = (v & 7) * 16
            d16 = st_d[gb, vr, pl.ds(vc, 16)]
            s16 = st_s[gb, vr, pl.ds(vc, 16)]
            m = (d16 >= mylo) & (d16 < myhi)
            cnt = plsc.all_reduce_population_count(m)
            idx = cur + _prefix16(m, iota) - 1
            plsc.store_scatter(csrc, [idx], s16, mask=m)
            plsc.store_scatter(cdst, [idx], d16 - mylo, mask=m)
            return cur + cnt

        cursor = lax.fori_loop(0, _SEG // 16, vreg_body, cursor)

        do_flush = (cursor[0] >= _FLUSH) | (g == nsegs - 1)

        @pl.when(do_flush & (cursor[0] > 0))
        def _():
            pend = (cursor + 15) & jnp.int32(-16)
            pidx = cursor + iota
            pm = pidx < pend
            plsc.store_scatter(csrc, [pidx], jnp.zeros((16,), jnp.int32),
                               mask=pm)
            plsc.store_scatter(cdst, [pidx], jnp.full((16,), _CT, jnp.int32),
                               mask=pm)
            flush(pend[0] >> 4)

        return jnp.where(do_flush, jnp.zeros((16,), jnp.int32), cursor)

    lax.fori_loop(0, nsegs, seg_body, jnp.zeros((16,), jnp.int32))


def _make_sc_degree(E_pad, n_out, npass):
    nsegs = E_pad // _SEG

    @functools.partial(
        pl.kernel,
        out_type=jax.ShapeDtypeStruct((n_out, 16), jnp.float32),
        mesh=_mesh(),
        scratch_types=[
            pltpu.VMEM((2, _SEG // 128, 128), jnp.int32),
            pltpu.VMEM((2, _SEG // 128, 128), jnp.int32),
            pltpu.VMEM((_CAP,), jnp.int32),
            pltpu.VMEM((_CAP,), jnp.int32),
            pltpu.VMEM((_CT + 8, 16), jnp.float32),
            pltpu.SemaphoreType.DMA,
        ],
        compiler_params=pltpu.CompilerParams(needs_layout_passes=False),
    )
    def deg_kernel(src2d, dst2d, out, st_s, st_d, csrc, cdst, deg, sem_s):
        c = lax.axis_index("c")
        s = lax.axis_index("s")
        w = s * 2 + c
        iota = lax.iota(jnp.int32, 16)
        ones = jnp.ones((16,), jnp.float32)

        for p in range(npass):
            mylo = (p * 32 + w) * _CT
            myhi = mylo + _CT

            def zbody(t, carry):
                deg[t, :] = jnp.zeros((16,), jnp.float32)
                return carry

            lax.fori_loop(0, _CT + 8, zbody, 0)

            def flush(nb):
                def bat(b, carry):
                    cd16 = cdst[pl.ds(b * 16, 16)]
                    for q in range(16):
                        r = cd16[q]
                        deg[r, :] = deg[r, :] + ones
                    return carry

                lax.fori_loop(0, nb, bat, 0)

            _scan_loop(nsegs, src2d, dst2d, st_s, st_d, csrc, cdst, sem_s,
                       mylo, myhi, iota, flush)

            pltpu.sync_copy(deg.at[pl.ds(0, _CT)],
                            out.at[pl.ds(mylo, _CT)])

    return deg_kernel


def _make_sc_segsum(E_pad, n_out, npass):
    nsegs = E_pad // _SEG

    @functools.partial(
        pl.kernel,
        out_type=jax.ShapeDtypeStruct((n_out, _H), jnp.float32),
        mesh=_mesh(),
        scratch_types=[
            pltpu.VMEM((2, _SEG // 128, 128), jnp.int32),
            pltpu.VMEM((2, _SEG // 128, 128), jnp.int32),
            pltpu.VMEM((_CAP,), jnp.int32),
            pltpu.VMEM((_CAP,), jnp.int32),
            pltpu.VMEM((_CT + 8, _H), jnp.float32),
            pltpu.VMEM((2, 16, _H), jnp.float32),
            pltpu.SemaphoreType.DMA,
            pltpu.SemaphoreType.DMA,
        ],
        compiler_params=pltpu.CompilerParams(needs_layout_passes=False),
    )
    def seg_kernel(hp, src2d, dst2d, out,
                   st_s, st_d, csrc, cdst, acc, rows, sem_s, sem_g):
        c = lax.axis_index("c")
        s = lax.axis_index("s")
        w = s * 2 + c
        iota = lax.iota(jnp.int32, 16)

        for p in range(npass):
            mylo = (p * 32 + w) * _CT
            myhi = mylo + _CT

            def zbody(t, carry):
                r = t >> 4
                k = t & 15
                acc[r, pl.ds(k * 16, 16)] = jnp.zeros((16,), jnp.float32)
                return carry

            lax.fori_loop(0, (_CT + 8) * 16, zbody, 0)

            def gather(b, jb):
                cs16 = csrc[pl.ds(b * 16, 16)]
                pltpu.async_copy(hp.at[cs16], rows.at[jb], sem_g)

            def gather_wait(b, jb):
                cs16 = csrc[pl.ds(b * 16, 16)]
                pltpu.make_async_copy(hp.at[cs16], rows.at[jb],
                                      sem_g).wait()

            def flush(nb):
                gather(0, 0)

                def bat(b, carry):
                    jb = lax.rem(b, 2)
                    gather_wait(b, jb)

                    @pl.when(b + 1 < nb)
                    def _():
                        gather(b + 1, 1 - jb)

                    cd16 = cdst[pl.ds(b * 16, 16)]
                    for q in range(16):
                        r = cd16[q]
                        for f in range(_H // 16):
                            sl = pl.ds(f * 16, 16)
                            acc[r, sl] = acc[r, sl] + rows[jb, q, sl]
                    return carry

                lax.fori_loop(0, nb, bat, 0)

            _scan_loop(nsegs, src2d, dst2d, st_s, st_d, csrc, cdst, sem_s,
                       mylo, myhi, iota, flush)

            pltpu.sync_copy(acc.at[pl.ds(0, _CT)],
                            out.at[pl.ds(mylo, _CT)])

    return seg_kernel


def _dinv_block(dp_ref):
    return lax.rsqrt(1.0 + dp_ref[:, 0:1])


def _tc_prep(x, W, degp):
    n_pad, D = x.shape

    def body(x_ref, w_ref, dp_ref, hp_ref):
        dinv = _dinv_block(dp_ref)
        h = jnp.dot(x_ref[...], w_ref[...], preferred_element_type=jnp.float32)
        hp_ref[...] = h * dinv

    return pl.pallas_call(
        body,
        grid=(n_pad // 256,),
        in_specs=[
            pl.BlockSpec((256, D), lambda i: (i, 0)),
            pl.BlockSpec((D, _H), lambda i: (0, 0)),
            pl.BlockSpec((256, 16), lambda i: (i, 0)),
        ],
        out_specs=pl.BlockSpec((256, _H), lambda i: (i, 0)),
        out_shape=jax.ShapeDtypeStruct((n_pad, _H), jnp.float32),
    )(x, W, degp)


def _tc_mid(acc, hp, degp, W2, b1):
    n_pad = hp.shape[0]

    def body(a_ref, h_ref, dp_ref, w_ref, b_ref, o_ref):
        dinv = _dinv_block(dp_ref)
        z = dinv * (a_ref[...] + h_ref[...]) + b_ref[...]
        a = jnp.maximum(z, 0.0)
        o_ref[...] = dinv * jnp.dot(a, w_ref[...],
                                    preferred_element_type=jnp.float32)

    return pl.pallas_call(
        body,
        grid=(n_pad // 256,),
        in_specs=[
            pl.BlockSpec((256, _H), lambda i: (i, 0)),
            pl.BlockSpec((256, _H), lambda i: (i, 0)),
            pl.BlockSpec((256, 16), lambda i: (i, 0)),
            pl.BlockSpec((_H, _H), lambda i: (0, 0)),
            pl.BlockSpec((1, _H), lambda i: (0, 0)),
        ],
        out_specs=pl.BlockSpec((256, _H), lambda i: (i, 0)),
        out_shape=jax.ShapeDtypeStruct((n_pad, _H), jnp.float32),
    )(acc, hp, degp, W2, b1)


def _tc_final(acc, hp, degp, b2):
    n_pad = hp.shape[0]

    def body(a_ref, h_ref, dp_ref, b_ref, o_ref):
        dinv = _dinv_block(dp_ref)
        o_ref[...] = dinv * (a_ref[...] + h_ref[...]) + b_ref[...]

    return pl.pallas_call(
        body,
        grid=(n_pad // 256,),
        in_specs=[
            pl.BlockSpec((256, _H), lambda i: (i, 0)),
            pl.BlockSpec((256, _H), lambda i: (i, 0)),
            pl.BlockSpec((256, 16), lambda i: (i, 0)),
            pl.BlockSpec((1, _H), lambda i: (0, 0)),
        ],
        out_specs=pl.BlockSpec((256, _H), lambda i: (i, 0)),
        out_shape=jax.ShapeDtypeStruct((n_pad, _H), jnp.float32),
    )(acc, hp, degp, b2)


def _tc_agg(x, W, b):
    n_pad, K = x.shape

    def body(x_ref, w_ref, b_ref, o_ref):
        o_ref[...] = jnp.dot(x_ref[...], w_ref[...],
                             preferred_element_type=jnp.float32) + b_ref[...]

    return pl.pallas_call(
        body,
        grid=(n_pad // 256,),
        in_specs=[
            pl.BlockSpec((256, K), lambda i: (i, 0)),
            pl.BlockSpec((K, _H), lambda i: (0, 0)),
            pl.BlockSpec((1, _H), lambda i: (0, 0)),
        ],
        out_specs=pl.BlockSpec((256, _H), lambda i: (i, 0)),
        out_shape=jax.ShapeDtypeStruct((n_pad, _H), jnp.float32),
    )(x, W, b)


def _encode(x, edge_index, W1, b1, W2, b2):
    N = x.shape[0]
    E = edge_index.shape[1]
    n_pad = _ceil_to(N, 256)
    npass = -(-n_pad // (32 * _CT))
    n_out = npass * 32 * _CT
    E_pad = _ceil_to(E, _SEG)

    src = edge_index[0].astype(jnp.int32)
    dst = edge_index[1].astype(jnp.int32)
    src2d = jnp.pad(src, (0, E_pad - E)).reshape(E_pad // 128, 128)
    dst2d = jnp.pad(dst, (0, E_pad - E),
                    constant_values=n_out).reshape(E_pad // 128, 128)

    xp = jnp.pad(x, ((0, n_pad - N), (0, 0)))
    b1_2d = b1.reshape(1, _H)
    b2_2d = b2.reshape(1, _H)

    deg = _make_sc_degree(E_pad, n_out, npass)(src2d, dst2d)
    seg = _make_sc_segsum(E_pad, n_out, npass)

    hp1 = _tc_prep(xp, W1, deg)
    acc1 = seg(hp1, src2d, dst2d)
    hp2 = _tc_mid(acc1, hp1, deg, W2, b1_2d)
    acc2 = seg(hp2, src2d, dst2d)
    out = _tc_final(acc2, hp2, deg, b2_2d)
    return out[:N]


def kernel(x_cell, x_gene, edge_index_cell, edge_index_gene,
           cell_W1, cell_b1, cell_W2, cell_b2,
           gene_W1, gene_b1, gene_W2, gene_b2,
           agg_W, agg_b):
    gene_embeddings = _encode(x_gene, edge_index_gene,
                              gene_W1, gene_b1, gene_W2, gene_b2)
    z_cell = _encode(x_cell, edge_index_cell,
                     cell_W1, cell_b1, cell_W2, cell_b2)

    N_cell = x_cell.shape[0]
    ge_r = gene_embeddings.reshape(N_cell, _SUB * _H)
    n_pad = _ceil_to(N_cell, 256)
    ge_rp = jnp.pad(ge_r, ((0, n_pad - N_cell), (0, 0)))
    z_gene = _tc_agg(ge_rp, agg_W, agg_b.reshape(1, _H))[:N_cell]

    z_combined = jnp.concatenate((z_cell, z_gene), axis=1)
    return (z_combined, z_cell, z_gene, gene_embeddings)


# speedup vs baseline: 2.1905x; 1.1834x over previous
import functools

import jax
import jax.numpy as jnp
from jax import lax
from jax.experimental import pallas as pl
from jax.experimental.pallas import tpu as pltpu
from jax.experimental.pallas import tpu_sc as plsc

_SUB = 5
_H = 256
_CT = 400
_SEG = 1024
_FLUSH = 2048
_CAP = 3072 + 32


def _mesh():
    return plsc.VectorSubcoreMesh(core_axis_name="c", subcore_axis_name="s")


def _ceil_to(x, m):
    return -(-x // m) * m


def _vgather(v, idx):
    dnums = lax.GatherDimensionNumbers(offset_dims=(), collapsed_slice_dims=(0,),
                                       start_index_map=(0,))
    return lax.gather(v, idx[:, None], dnums, slice_sizes=(1,),
                      mode=lax.GatherScatterMode.PROMISE_IN_BOUNDS)


def _prefix16(m, iota):
    cs = jnp.where(m, 1, 0)
    for sh in (1, 2, 4, 8):
        shifted = _vgather(cs, jnp.maximum(iota - sh, 0))
        cs = cs + jnp.where(iota >= sh, shifted, 0)
    return cs


def _scan_loop(nsegs, src2d, dst2d, st_s, st_d, csrc, cdst, sem_s,
               iota, compact_fn, flush, store_src, trash):
    def stage(g, gb):
        pltpu.async_copy(src2d.at[pl.ds(g * (_SEG // 128), _SEG // 128)],
                         st_s.at[gb], sem_s)
        pltpu.async_copy(dst2d.at[pl.ds(g * (_SEG // 128), _SEG // 128)],
                         st_d.at[gb], sem_s)

    def stage_wait(g, gb):
        pltpu.make_async_copy(src2d.at[pl.ds(g * (_SEG // 128), _SEG // 128)],
                              st_s.at[gb], sem_s).wait()
        pltpu.make_async_copy(dst2d.at[pl.ds(g * (_SEG // 128), _SEG // 128)],
                              st_d.at[gb], sem_s).wait()

    stage(0, 0)

    def seg_body(g, cursor):
        gb = lax.rem(g, 2)
        stage_wait(g, gb)

        @pl.when(g + 1 < nsegs)
        def _():
            stage(g + 1, lax.rem(g + 1, 2))

        def vreg_body(v, cur):
            for k in range(4):
                v4 = v * 4 + k
                vr = v4 >> 3
                vc = (v4 & 7) * 16
                d16 = st_d[gb, vr, pl.ds(vc, 16)]
                m, rel = compact_fn(d16)
                cnt = plsc.all_reduce_population_count(m)
                idx = cur + _prefix16(m, iota) - 1
                if store_src:
                    s16 = st_s[gb, vr, pl.ds(vc, 16)]
                    plsc.store_scatter(csrc, [idx], s16, mask=m)
                plsc.store_scatter(cdst, [idx], rel, mask=m)
                cur = cur + cnt
            return cur

        cursor = lax.fori_loop(0, _SEG // 64, vreg_body, cursor)

        do_flush = (cursor[0] >= _FLUSH) | (g == nsegs - 1)

        @pl.when(do_flush & (cursor[0] > 0))
        def _():
            pend = (cursor + 15) & jnp.int32(-16)
            pidx = cursor + iota
            pm = pidx < pend
            if store_src:
                plsc.store_scatter(csrc, [pidx], jnp.zeros((16,), jnp.int32),
                                   mask=pm)
            plsc.store_scatter(cdst, [pidx], jnp.full((16,), trash, jnp.int32),
                               mask=pm)
            flush(pend[0] >> 4)

        return jnp.where(do_flush, jnp.zeros((16,), jnp.int32), cursor)

    lax.fori_loop(0, nsegs, seg_body, jnp.zeros((16,), jnp.int32))


def _make_sc_degree(E_pad, n_outd):
    nsegs = E_pad // _SEG
    CTD = n_outd // 32
    CTP = CTD + 64
    trash = CTD

    @functools.partial(
        pl.kernel,
        out_type=jax.ShapeDtypeStruct((n_outd // 8, 128), jnp.float32),
        mesh=_mesh(),
        scratch_types=[
            pltpu.VMEM((2, _SEG // 128, 128), jnp.int32),
            pltpu.VMEM((2, _SEG // 128, 128), jnp.int32),
            pltpu.VMEM((16,), jnp.int32),
            pltpu.VMEM((_CAP,), jnp.int32),
            pltpu.VMEM((CTP // 8, 128), jnp.float32),
            pltpu.SemaphoreType.DMA,
        ],
        compiler_params=pltpu.CompilerParams(needs_layout_passes=False),
    )
    def deg_kernel(src2d, dst2d, out, st_s, st_d, csrc, cdst, deg, sem_s):
        c = lax.axis_index("c")
        s = lax.axis_index("s")
        w = s * 2 + c
        iota = lax.iota(jnp.int32, 16)
        ones = jnp.ones((16,), jnp.float32)
        mylo = w * CTD

        def zbody(t, carry):
            deg[t >> 3, pl.ds((t & 7) * 16, 16)] = jnp.zeros((16,),
                                                             jnp.float32)
            return carry

        lax.fori_loop(0, CTP, zbody, 0)

        def compact_fn(d16):
            m = (d16 >= mylo) & (d16 < mylo + CTD)
            return m, d16 - mylo

        def flush(nb):
            def bat(b, carry):
                cd16 = cdst[pl.ds(b * 16, 16)]
                for q in range(16):
                    r = cd16[q]
                    rr = r >> 3
                    rc = (r & 7) * 16
                    deg[rr, pl.ds(rc, 16)] = deg[rr, pl.ds(rc, 16)] + ones
                return carry

            lax.fori_loop(0, nb, bat, 0)

        _scan_loop(nsegs, src2d, dst2d, st_s, st_d, csrc, cdst, sem_s,
                   iota, compact_fn, flush, False, trash)

        pltpu.sync_copy(deg.at[pl.ds(0, CTD // 8)],
                        out.at[pl.ds(w * (CTD // 8), CTD // 8)])

    return deg_kernel


def _make_sc_segsum(E_pad, n_out, npass):
    nsegs = E_pad // _SEG

    @functools.partial(
        pl.kernel,
        out_type=jax.ShapeDtypeStruct((n_out, _H), jnp.float32),
        mesh=_mesh(),
        scratch_types=[
            pltpu.VMEM((2, _SEG // 128, 128), jnp.int32),
            pltpu.VMEM((2, _SEG // 128, 128), jnp.int32),
            pltpu.VMEM((_CAP,), jnp.int32),
            pltpu.VMEM((_CAP,), jnp.int32),
            pltpu.VMEM((_CT + 8, _H), jnp.float32),
            pltpu.VMEM((2, 16, _H), jnp.float32),
            pltpu.SemaphoreType.DMA,
            pltpu.SemaphoreType.DMA,
        ],
        compiler_params=pltpu.CompilerParams(needs_layout_passes=False),
    )
    def seg_kernel(hp, src2d, dst2d, out,
                   st_s, st_d, csrc, cdst, acc, rows, sem_s, sem_g):
        c = lax.axis_index("c")
        s = lax.axis_index("s")
        w = s * 2 + c
        iota = lax.iota(jnp.int32, 16)

        for p in range(npass):
            mylo = (p * 32 + w) * _CT
            myhi = mylo + _CT

            def zbody(t, carry):
                r = t >> 4
                k = t & 15
                acc[r, pl.ds(k * 16, 16)] = jnp.zeros((16,), jnp.float32)
                return carry

            lax.fori_loop(0, (_CT + 8) * 16, zbody, 0)

            def gather(b, jb):
                cs16 = csrc[pl.ds(b * 16, 16)]
                pltpu.async_copy(hp.at[cs16], rows.at[jb], sem_g)

            def gather_wait(b, jb):
                cs16 = csrc[pl.ds(b * 16, 16)]
                pltpu.make_async_copy(hp.at[cs16], rows.at[jb],
                                      sem_g).wait()

            def flush(nb):
                gather(0, 0)

                def bat(b, carry):
                    jb = lax.rem(b, 2)
                    gather_wait(b, jb)

                    @pl.when(b + 1 < nb)
                    def _():
                        gather(b + 1, 1 - jb)

                    cd16 = cdst[pl.ds(b * 16, 16)]
                    for q in range(16):
                        r = cd16[q]
                        for f in range(_H // 16):
                            sl = pl.ds(f * 16, 16)
                            acc[r, sl] = acc[r, sl] + rows[jb, q, sl]
                    return carry

                lax.fori_loop(0, nb, bat, 0)

            def compact_fn(d16):
                m = (d16 >= mylo) & (d16 < myhi)
                return m, d16 - mylo

            _scan_loop(nsegs, src2d, dst2d, st_s, st_d, csrc, cdst, sem_s,
                       iota, compact_fn, flush, True, _CT)

            pltpu.sync_copy(acc.at[pl.ds(0, _CT)],
                            out.at[pl.ds(mylo, _CT)])

    return seg_kernel


def _dinv_block(dp_ref):
    return lax.rsqrt(1.0 + dp_ref[:, 0:1])


def _tc_prep(x, W, degp):
    n_pad, D = x.shape

    def body(x_ref, w_ref, dp_ref, hp_ref):
        dinv = _dinv_block(dp_ref)
        h = jnp.dot(x_ref[...], w_ref[...], preferred_element_type=jnp.float32)
        hp_ref[...] = h * dinv

    return pl.pallas_call(
        body,
        grid=(n_pad // 256,),
        in_specs=[
            pl.BlockSpec((256, D), lambda i: (i, 0)),
            pl.BlockSpec((D, _H), lambda i: (0, 0)),
            pl.BlockSpec((256, 16), lambda i: (i, 0)),
        ],
        out_specs=pl.BlockSpec((256, _H), lambda i: (i, 0)),
        out_shape=jax.ShapeDtypeStruct((n_pad, _H), jnp.float32),
    )(x, W, degp)


def _tc_mid(acc, hp, degp, W2, b1):
    n_pad = hp.shape[0]

    def body(a_ref, h_ref, dp_ref, w_ref, b_ref, o_ref):
        dinv = _dinv_block(dp_ref)
        z = dinv * (a_ref[...] + h_ref[...]) + b_ref[...]
        a = jnp.maximum(z, 0.0)
        o_ref[...] = dinv * jnp.dot(a, w_ref[...],
                                    preferred_element_type=jnp.float32)

    return pl.pallas_call(
        body,
        grid=(n_pad // 256,),
        in_specs=[
            pl.BlockSpec((256, _H), lambda i: (i, 0)),
            pl.BlockSpec((256, _H), lambda i: (i, 0)),
            pl.BlockSpec((256, 16), lambda i: (i, 0)),
            pl.BlockSpec((_H, _H), lambda i: (0, 0)),
            pl.BlockSpec((1, _H), lambda i: (0, 0)),
        ],
        out_specs=pl.BlockSpec((256, _H), lambda i: (i, 0)),
        out_shape=jax.ShapeDtypeStruct((n_pad, _H), jnp.float32),
    )(acc, hp, degp, W2, b1)


def _tc_final(acc, hp, degp, b2):
    n_pad = hp.shape[0]

    def body(a_ref, h_ref, dp_ref, b_ref, o_ref):
        dinv = _dinv_block(dp_ref)
        o_ref[...] = dinv * (a_ref[...] + h_ref[...]) + b_ref[...]

    return pl.pallas_call(
        body,
        grid=(n_pad // 256,),
        in_specs=[
            pl.BlockSpec((256, _H), lambda i: (i, 0)),
            pl.BlockSpec((256, _H), lambda i: (i, 0)),
            pl.BlockSpec((256, 16), lambda i: (i, 0)),
            pl.BlockSpec((1, _H), lambda i: (0, 0)),
        ],
        out_specs=pl.BlockSpec((256, _H), lambda i: (i, 0)),
        out_shape=jax.ShapeDtypeStruct((n_pad, _H), jnp.float32),
    )(acc, hp, degp, b2)


def _tc_agg(x, W, b):
    n_pad, K = x.shape

    def body(x_ref, w_ref, b_ref, o_ref):
        o_ref[...] = jnp.dot(x_ref[...], w_ref[...],
                             preferred_element_type=jnp.float32) + b_ref[...]

    return pl.pallas_call(
        body,
        grid=(n_pad // 256,),
        in_specs=[
            pl.BlockSpec((256, K), lambda i: (i, 0)),
            pl.BlockSpec((K, _H), lambda i: (0, 0)),
            pl.BlockSpec((1, _H), lambda i: (0, 0)),
        ],
        out_specs=pl.BlockSpec((256, _H), lambda i: (i, 0)),
        out_shape=jax.ShapeDtypeStruct((n_pad, _H), jnp.float32),
    )(x, W, b)


def _encode(x, edge_index, W1, b1, W2, b2):
    N = x.shape[0]
    E = edge_index.shape[1]
    n_pad = _ceil_to(N, 256)
    npass = -(-n_pad // (32 * _CT))
    n_out = npass * 32 * _CT
    E_pad = _ceil_to(E, _SEG)

    src = edge_index[0].astype(jnp.int32)
    dst = edge_index[1].astype(jnp.int32)
    src2d = jnp.pad(src, (0, E_pad - E)).reshape(E_pad // 128, 128)
    dst2d = jnp.pad(dst, (0, E_pad - E),
                    constant_values=n_out).reshape(E_pad // 128, 128)

    xp = jnp.pad(x, ((0, n_pad - N), (0, 0)))
    b1_2d = b1.reshape(1, _H)
    b2_2d = b2.reshape(1, _H)

    n_outd = _ceil_to(n_pad, 2048)
    deg = _make_sc_degree(E_pad, n_outd)(src2d, dst2d).reshape(n_outd, 16)
    seg = _make_sc_segsum(E_pad, n_out, npass)

    hp1 = _tc_prep(xp, W1, deg)
    acc1 = seg(hp1, src2d, dst2d)
    hp2 = _tc_mid(acc1, hp1, deg, W2, b1_2d)
    acc2 = seg(hp2, src2d, dst2d)
    out = _tc_final(acc2, hp2, deg, b2_2d)
    return out[:N]


def kernel(x_cell, x_gene, edge_index_cell, edge_index_gene,
           cell_W1, cell_b1, cell_W2, cell_b2,
           gene_W1, gene_b1, gene_W2, gene_b2,
           agg_W, agg_b):
    gene_embeddings = _encode(x_gene, edge_index_gene,
                              gene_W1, gene_b1, gene_W2, gene_b2)
    z_cell = _encode(x_cell, edge_index_cell,
                     cell_W1, cell_b1, cell_W2, cell_b2)

    N_cell = x_cell.shape[0]
    ge_r = gene_embeddings.reshape(N_cell, _SUB * _H)
    n_pad = _ceil_to(N_cell, 256)
    ge_rp = jnp.pad(ge_r, ((0, n_pad - N_cell), (0, 0)))
    z_gene = _tc_agg(ge_rp, agg_W, agg_b.reshape(1, _H))[:N_cell]

    z_combined = jnp.concatenate((z_cell, z_gene), axis=1)
    return (z_combined, z_cell, z_gene, gene_embeddings)


# speedup vs baseline: 2.8994x; 1.3236x over previous
import functools

import jax
import jax.numpy as jnp
from jax import lax
from jax.experimental import pallas as pl
from jax.experimental.pallas import tpu as pltpu
from jax.experimental.pallas import tpu_sc as plsc

_SUB = 5
_H = 256
_CT = 400
_SEG = 1024
_FLUSH = 2048
_CAP = 3072 + 32


def _mesh():
    return plsc.VectorSubcoreMesh(core_axis_name="c", subcore_axis_name="s")


def _ceil_to(x, m):
    return -(-x // m) * m


def _vgather(v, idx):
    dnums = lax.GatherDimensionNumbers(offset_dims=(), collapsed_slice_dims=(0,),
                                       start_index_map=(0,))
    return lax.gather(v, idx[:, None], dnums, slice_sizes=(1,),
                      mode=lax.GatherScatterMode.PROMISE_IN_BOUNDS)


def _prefix16(m, iota):
    cs = jnp.where(m, 1, 0)
    for sh in (1, 2, 4, 8):
        shifted = _vgather(cs, jnp.maximum(iota - sh, 0))
        cs = cs + jnp.where(iota >= sh, shifted, 0)
    return cs


def _scan_loop(nsegs, src2d, dst2d, st_s, st_d, csrc, cdst, sem_s,
               iota, compact_fn, flush, store_src, trash):
    def stage(g, gb):
        pltpu.async_copy(src2d.at[pl.ds(g * (_SEG // 128), _SEG // 128)],
                         st_s.at[gb], sem_s)
        pltpu.async_copy(dst2d.at[pl.ds(g * (_SEG // 128), _SEG // 128)],
                         st_d.at[gb], sem_s)

    def stage_wait(g, gb):
        pltpu.make_async_copy(src2d.at[pl.ds(g * (_SEG // 128), _SEG // 128)],
                              st_s.at[gb], sem_s).wait()
        pltpu.make_async_copy(dst2d.at[pl.ds(g * (_SEG // 128), _SEG // 128)],
                              st_d.at[gb], sem_s).wait()

    stage(0, 0)

    def seg_body(g, cursor):
        gb = lax.rem(g, 2)
        stage_wait(g, gb)

        @pl.when(g + 1 < nsegs)
        def _():
            stage(g + 1, lax.rem(g + 1, 2))

        def vreg_body(v, cur):
            lanes = []
            tot = None
            for k in range(4):
                v4 = v * 4 + k
                vr = v4 >> 3
                vc = (v4 & 7) * 16
                d16 = st_d[gb, vr, pl.ds(vc, 16)]
                m, rel = compact_fn(d16)
                cnt = plsc.all_reduce_population_count(m)
                lanes.append((vr, vc, m, rel, cnt))
                tot = cnt if tot is None else tot + cnt

            @pl.when(tot[0] > 0)
            def _():
                cc = cur
                for vr, vc, m, rel, cnt in lanes:
                    idx = cc + _prefix16(m, iota) - 1
                    if store_src:
                        s16 = st_s[gb, vr, pl.ds(vc, 16)]
                        plsc.store_scatter(csrc, [idx], s16, mask=m)
                    plsc.store_scatter(cdst, [idx], rel, mask=m)
                    cc = cc + cnt

            return cur + tot

        cursor = lax.fori_loop(0, _SEG // 64, vreg_body, cursor)

        do_flush = (cursor[0] >= _FLUSH) | (g == nsegs - 1)

        @pl.when(do_flush & (cursor[0] > 0))
        def _():
            pend = (cursor + 15) & jnp.int32(-16)
            pidx = cursor + iota
            pm = pidx < pend
            if store_src:
                plsc.store_scatter(csrc, [pidx], jnp.zeros((16,), jnp.int32),
                                   mask=pm)
            plsc.store_scatter(cdst, [pidx], jnp.full((16,), trash, jnp.int32),
                               mask=pm)
            flush(pend[0] >> 4)

        return jnp.where(do_flush, jnp.zeros((16,), jnp.int32), cursor)

    lax.fori_loop(0, nsegs, seg_body, jnp.zeros((16,), jnp.int32))


def _make_sc_degree(E_pad, n_outd):
    nsegs = E_pad // _SEG
    CTD = n_outd // 32
    CTP = CTD + 64
    trash = CTD

    @functools.partial(
        pl.kernel,
        out_type=jax.ShapeDtypeStruct((n_outd // 8, 128), jnp.float32),
        mesh=_mesh(),
        scratch_types=[
            pltpu.VMEM((2, _SEG // 128, 128), jnp.int32),
            pltpu.VMEM((2, _SEG // 128, 128), jnp.int32),
            pltpu.VMEM((16,), jnp.int32),
            pltpu.VMEM((_CAP,), jnp.int32),
            pltpu.VMEM((CTP // 8, 128), jnp.float32),
            pltpu.SemaphoreType.DMA,
        ],
        compiler_params=pltpu.CompilerParams(needs_layout_passes=False),
    )
    def deg_kernel(src2d, dst2d, out, st_s, st_d, csrc, cdst, deg, sem_s):
        c = lax.axis_index("c")
        s = lax.axis_index("s")
        w = s * 2 + c
        iota = lax.iota(jnp.int32, 16)
        ones = jnp.ones((16,), jnp.float32)
        mylo = w * CTD

        def zbody(t, carry):
            deg[t >> 3, pl.ds((t & 7) * 16, 16)] = jnp.zeros((16,),
                                                             jnp.float32)
            return carry

        lax.fori_loop(0, CTP, zbody, 0)

        def compact_fn(d16):
            m = (d16 >= mylo) & (d16 < mylo + CTD)
            return m, d16 - mylo

        def flush(nb):
            def bat(b, carry):
                cd16 = cdst[pl.ds(b * 16, 16)]
                for q in range(16):
                    r = cd16[q]
                    rr = r >> 3
                    rc = (r & 7) * 16
                    deg[rr, pl.ds(rc, 16)] = deg[rr, pl.ds(rc, 16)] + ones
                return carry

            lax.fori_loop(0, nb, bat, 0)

        _scan_loop(nsegs, src2d, dst2d, st_s, st_d, csrc, cdst, sem_s,
                   iota, compact_fn, flush, False, trash)

        pltpu.sync_copy(deg.at[pl.ds(0, CTD // 8)],
                        out.at[pl.ds(w * (CTD // 8), CTD // 8)])

    return deg_kernel


def _make_sc_segsum(E_pad, n_out, npass):
    nsegs = E_pad // _SEG

    @functools.partial(
        pl.kernel,
        out_type=jax.ShapeDtypeStruct((n_out, _H), jnp.float32),
        mesh=_mesh(),
        scratch_types=[
            pltpu.VMEM((2, _SEG // 128, 128), jnp.int32),
            pltpu.VMEM((2, _SEG // 128, 128), jnp.int32),
            pltpu.VMEM((_CAP,), jnp.int32),
            pltpu.VMEM((_CAP,), jnp.int32),
            pltpu.VMEM((_CT + 8, _H), jnp.float32),
            pltpu.VMEM((3, 16, _H), jnp.float32),
            pltpu.SemaphoreType.DMA,
            pltpu.SemaphoreType.DMA,
        ],
        compiler_params=pltpu.CompilerParams(needs_layout_passes=False),
    )
    def seg_kernel(hp, src2d, dst2d, out,
                   st_s, st_d, csrc, cdst, acc, rows, sem_s, sem_g):
        c = lax.axis_index("c")
        s = lax.axis_index("s")
        w = s * 2 + c
        iota = lax.iota(jnp.int32, 16)

        for p in range(npass):
            mylo = (p * 32 + w) * _CT
            myhi = mylo + _CT

            def zbody(t, carry):
                r = t >> 4
                k = t & 15
                acc[r, pl.ds(k * 16, 16)] = jnp.zeros((16,), jnp.float32)
                return carry

            lax.fori_loop(0, (_CT + 8) * 16, zbody, 0)

            def gather(b, jb):
                cs16 = csrc[pl.ds(b * 16, 16)]
                pltpu.async_copy(hp.at[cs16], rows.at[jb], sem_g)

            def gather_wait(b, jb):
                cs16 = csrc[pl.ds(b * 16, 16)]
                pltpu.make_async_copy(hp.at[cs16], rows.at[jb],
                                      sem_g).wait()

            def flush(nb):
                for i in range(3):
                    @pl.when(i < nb)
                    def _(i=i):
                        gather(i, i)

                def bat(b, carry):
                    jb = lax.rem(b, 3)
                    gather_wait(b, jb)

                    cd16 = cdst[pl.ds(b * 16, 16)]
                    for q in range(16):
                        r = cd16[q]
                        for f in range(_H // 16):
                            sl = pl.ds(f * 16, 16)
                            acc[r, sl] = acc[r, sl] + rows[jb, q, sl]

                    @pl.when(b + 3 < nb)
                    def _():
                        gather(b + 3, jb)

                    return carry

                lax.fori_loop(0, nb, bat, 0)

            def compact_fn(d16):
                m = (d16 >= mylo) & (d16 < myhi)
                return m, d16 - mylo

            _scan_loop(nsegs, src2d, dst2d, st_s, st_d, csrc, cdst, sem_s,
                       iota, compact_fn, flush, True, _CT)

            pltpu.sync_copy(acc.at[pl.ds(0, _CT)],
                            out.at[pl.ds(mylo, _CT)])

    return seg_kernel


def _dinv_block(dp_ref):
    return lax.rsqrt(1.0 + dp_ref[:, 0:1])


def _tc_prep(x, W, degp):
    n_pad, D = x.shape

    def body(x_ref, w_ref, dp_ref, hp_ref):
        dinv = _dinv_block(dp_ref)
        h = jnp.dot(x_ref[...], w_ref[...], preferred_element_type=jnp.float32)
        hp_ref[...] = h * dinv

    return pl.pallas_call(
        body,
        grid=(n_pad // 256,),
        in_specs=[
            pl.BlockSpec((256, D), lambda i: (i, 0)),
            pl.BlockSpec((D, _H), lambda i: (0, 0)),
            pl.BlockSpec((256, 16), lambda i: (i, 0)),
        ],
        out_specs=pl.BlockSpec((256, _H), lambda i: (i, 0)),
        out_shape=jax.ShapeDtypeStruct((n_pad, _H), jnp.float32),
    )(x, W, degp)


def _tc_mid(acc, hp, degp, W2, b1):
    n_pad = hp.shape[0]

    def body(a_ref, h_ref, dp_ref, w_ref, b_ref, o_ref):
        dinv = _dinv_block(dp_ref)
        z = dinv * (a_ref[...] + h_ref[...]) + b_ref[...]
        a = jnp.maximum(z, 0.0)
        o_ref[...] = dinv * jnp.dot(a, w_ref[...],
                                    preferred_element_type=jnp.float32)

    return pl.pallas_call(
        body,
        grid=(n_pad // 256,),
        in_specs=[
            pl.BlockSpec((256, _H), lambda i: (i, 0)),
            pl.BlockSpec((256, _H), lambda i: (i, 0)),
            pl.BlockSpec((256, 16), lambda i: (i, 0)),
            pl.BlockSpec((_H, _H), lambda i: (0, 0)),
            pl.BlockSpec((1, _H), lambda i: (0, 0)),
        ],
        out_specs=pl.BlockSpec((256, _H), lambda i: (i, 0)),
        out_shape=jax.ShapeDtypeStruct((n_pad, _H), jnp.float32),
    )(acc, hp, degp, W2, b1)


def _tc_final(acc, hp, degp, b2):
    n_pad = hp.shape[0]

    def body(a_ref, h_ref, dp_ref, b_ref, o_ref):
        dinv = _dinv_block(dp_ref)
        o_ref[...] = dinv * (a_ref[...] + h_ref[...]) + b_ref[...]

    return pl.pallas_call(
        body,
        grid=(n_pad // 256,),
        in_specs=[
            pl.BlockSpec((256, _H), lambda i: (i, 0)),
            pl.BlockSpec((256, _H), lambda i: (i, 0)),
            pl.BlockSpec((256, 16), lambda i: (i, 0)),
            pl.BlockSpec((1, _H), lambda i: (0, 0)),
        ],
        out_specs=pl.BlockSpec((256, _H), lambda i: (i, 0)),
        out_shape=jax.ShapeDtypeStruct((n_pad, _H), jnp.float32),
    )(acc, hp, degp, b2)


def _tc_agg(x, W, b):
    n_pad, K = x.shape

    def body(x_ref, w_ref, b_ref, o_ref):
        o_ref[...] = jnp.dot(x_ref[...], w_ref[...],
                             preferred_element_type=jnp.float32) + b_ref[...]

    return pl.pallas_call(
        body,
        grid=(n_pad // 256,),
        in_specs=[
            pl.BlockSpec((256, K), lambda i: (i, 0)),
            pl.BlockSpec((K, _H), lambda i: (0, 0)),
            pl.BlockSpec((1, _H), lambda i: (0, 0)),
        ],
        out_specs=pl.BlockSpec((256, _H), lambda i: (i, 0)),
        out_shape=jax.ShapeDtypeStruct((n_pad, _H), jnp.float32),
    )(x, W, b)


def _encode(x, edge_index, W1, b1, W2, b2):
    N = x.shape[0]
    E = edge_index.shape[1]
    n_pad = _ceil_to(N, 256)
    npass = -(-n_pad // (32 * _CT))
    n_out = npass * 32 * _CT
    E_pad = _ceil_to(E, _SEG)

    src = edge_index[0].astype(jnp.int32)
    dst = edge_index[1].astype(jnp.int32)
    src2d = jnp.pad(src, (0, E_pad - E)).reshape(E_pad // 128, 128)
    dst2d = jnp.pad(dst, (0, E_pad - E),
                    constant_values=n_out).reshape(E_pad // 128, 128)

    xp = jnp.pad(x, ((0, n_pad - N), (0, 0)))
    b1_2d = b1.reshape(1, _H)
    b2_2d = b2.reshape(1, _H)

    n_outd = _ceil_to(n_pad, 2048)
    deg = _make_sc_degree(E_pad, n_outd)(src2d, dst2d).reshape(n_outd, 16)
    seg = _make_sc_segsum(E_pad, n_out, npass)

    hp1 = _tc_prep(xp, W1, deg)
    acc1 = seg(hp1, src2d, dst2d)
    hp2 = _tc_mid(acc1, hp1, deg, W2, b1_2d)
    acc2 = seg(hp2, src2d, dst2d)
    out = _tc_final(acc2, hp2, deg, b2_2d)
    return out[:N]


def kernel(x_cell, x_gene, edge_index_cell, edge_index_gene,
           cell_W1, cell_b1, cell_W2, cell_b2,
           gene_W1, gene_b1, gene_W2, gene_b2,
           agg_W, agg_b):
    gene_embeddings = _encode(x_gene, edge_index_gene,
                              gene_W1, gene_b1, gene_W2, gene_b2)
    z_cell = _encode(x_cell, edge_index_cell,
                     cell_W1, cell_b1, cell_W2, cell_b2)

    N_cell = x_cell.shape[0]
    ge_r = gene_embeddings.reshape(N_cell, _SUB * _H)
    n_pad = _ceil_to(N_cell, 256)
    ge_rp = jnp.pad(ge_r, ((0, n_pad - N_cell), (0, 0)))
    z_gene = _tc_agg(ge_rp, agg_W, agg_b.reshape(1, _H))[:N_cell]

    z_combined = jnp.concatenate((z_cell, z_gene), axis=1)
    return (z_combined, z_cell, z_gene, gene_embeddings)
